# Initial kernel scaffold; baseline (speedup 1.0000x reference)
import jax
import jax.numpy as jnp
from jax.experimental import pallas as pl


def kernel(user_item, user_cat, item_cat, linear_W, linear_b, u_table, i_table, uf_table, if_table, bn_g, bn_b, W1, b1, g1, bt1, W2, b2, g2, bt2, W3, b3):
    raise NotImplementedError("write your pallas kernel here")


# speedup vs baseline: 2.5244x; 2.5244x over previous
import jax
import jax.numpy as jnp
import numpy as np
from jax import lax
from jax.experimental import pallas as pl
from jax.experimental.pallas import tpu as pltpu
from jax.experimental.pallas import tpu_sc as plsc

EPS = 1e-5
NC = 2
NS = 16
NW = NC * NS
CH = 128


def _sc_body(user_h, item_h, uf_h, if_h, lincat_h, uirow_h, uilane_h,
             u_tab, i_tab, uf_tab, if_tab, wlin2_h, lincat_tab_h,
             cross_out, lin_out,
             cat_tab_v, u_idx_v, i_idx_v, uf_idx_v, if_idx_v,
             lincat_v, uirow_v, uilane_v,
             emb_rows_v, ui_rows_v, out_cross_v, lin_v, sem):
    wid = lax.axis_index("s") * NC + lax.axis_index("c")
    pw = u_idx_v.shape[0]
    n_chunks = pw // CH
    base = wid * pw
    n_uf = uf_idx_v.shape[0]
    n_if = if_idx_v.shape[0]
    nf = 2 + n_uf + n_if

    pltpu.sync_copy(lincat_tab_h, cat_tab_v)
    pltpu.sync_copy(user_h.at[pl.ds(base, pw)], u_idx_v)
    pltpu.sync_copy(item_h.at[pl.ds(base, pw)], i_idx_v)
    for f in range(n_uf):
        pltpu.sync_copy(uf_h.at[f, pl.ds(base, pw)], uf_idx_v.at[f])
    for f in range(n_if):
        pltpu.sync_copy(if_h.at[f, pl.ds(base, pw)], if_idx_v.at[f])
    for f in range(n_uf + n_if):
        pltpu.sync_copy(lincat_h.at[f, pl.ds(base, pw)], lincat_v.at[f])
    for f in range(2):
        pltpu.sync_copy(uirow_h.at[f, pl.ds(base, pw)], uirow_v.at[f])
        pltpu.sync_copy(uilane_h.at[f, pl.ds(base, pw)], uilane_v.at[f])

    lane_iota = lax.broadcasted_iota(jnp.int32, (16,), 0)

    for c in range(n_chunks):
        o = c * CH
        copies = [
            pltpu.async_copy(u_tab.at[u_idx_v.at[pl.ds(o, CH)]],
                             emb_rows_v.at[pl.ds(0, CH)], sem),
            pltpu.async_copy(i_tab.at[i_idx_v.at[pl.ds(o, CH)]],
                             emb_rows_v.at[pl.ds(CH, CH)], sem),
        ]
        for f in range(n_uf):
            copies.append(pltpu.async_copy(
                uf_tab.at[uf_idx_v.at[f, pl.ds(o, CH)]],
                emb_rows_v.at[pl.ds((2 + f) * CH, CH)], sem))
        for f in range(n_if):
            copies.append(pltpu.async_copy(
                if_tab.at[if_idx_v.at[f, pl.ds(o, CH)]],
                emb_rows_v.at[pl.ds((2 + n_uf + f) * CH, CH)], sem))
        for f in range(2):
            copies.append(pltpu.async_copy(
                wlin2_h.at[uirow_v.at[f, pl.ds(o, CH)]],
                ui_rows_v.at[pl.ds(f * CH, CH)], sem))
        for cp in copies:
            cp.wait()

        def fm_row(l, carry):
            s = emb_rows_v[l]
            q = s * s
            for f in range(1, nf):
                e = emb_rows_v[f * CH + l]
                s = s + e
                q = q + e * e
            out_cross_v[l] = 0.5 * (s * s - q)
            return carry

        lax.fori_loop(0, CH, fm_row, 0)

        def lin_grp(g, carry):
            off = o + g * 16
            acc = jnp.zeros((16,), jnp.float32)
            for f in range(n_uf + n_if):
                iv = lincat_v[f, pl.ds(off, 16)]
                acc = acc + plsc.load_gather(cat_tab_v, [iv])
            rows_base = lane_iota + g * 16
            for f in range(2):
                lanev = uilane_v[f, pl.ds(off, 16)]
                acc = acc + plsc.load_gather(ui_rows_v,
                                             [rows_base + f * CH, lanev])
            lin_v[pl.ds(off, 16)] = acc
            return carry

        lax.fori_loop(0, CH // 16, lin_grp, 0)

        pltpu.sync_copy(out_cross_v, cross_out.at[pl.ds(base + o, CH)])

    pltpu.sync_copy(lin_v, lin_out.at[pl.ds(base, pw)])


def _tc_body(cross_ref, lin_ref, bn_g_ref, bn_b_ref, w1_ref, b1_ref, g1_ref,
             bt1_ref, w2_ref, b2_ref, g2_ref, bt2_ref, w3_ref, b3_ref,
             lb_ref, out_ref):
    def bn(v, g, b):
        mu = jnp.mean(v, axis=0, keepdims=True)
        vc = v - mu
        var = jnp.mean(vc * vc, axis=0, keepdims=True)
        return vc / jnp.sqrt(var + EPS) * g + b

    x = bn(cross_ref[...], bn_g_ref[...], bn_b_ref[...])
    h = jnp.dot(x, w1_ref[...], preferred_element_type=jnp.float32) + b1_ref[...]
    h = jnp.maximum(bn(h, g1_ref[...], bt1_ref[...]), 0.0)
    h = jnp.dot(h, w2_ref[...], preferred_element_type=jnp.float32) + b2_ref[...]
    h = jnp.maximum(bn(h, g2_ref[...], bt2_ref[...]), 0.0)
    z = jnp.dot(h, w3_ref[...], preferred_element_type=jnp.float32) + b3_ref[...]
    z = z + lin_ref[...] + lb_ref[...]
    out_ref[...] = 1.0 / (1.0 + jnp.exp(-z))


def kernel(user_item, user_cat, item_cat, linear_W, linear_b, u_table,
           i_table, uf_table, if_table, bn_g, bn_b, W1, b1, g1, bt1, W2, b2,
           g2, bt2, W3, b3):
    B = user_item.shape[0]
    D = u_table.shape[1]
    n_uf = user_cat.shape[1]
    n_if = item_cat.shape[1]
    UV = u_table.shape[0]
    IV = i_table.shape[0]
    CF = uf_table.shape[0] // n_uf
    pw = B // NW
    total_lin = linear_W.shape[0]

    user = user_item[:, 0].astype(jnp.int32)
    item = user_item[:, 1].astype(jnp.int32)
    uf_off = jnp.asarray((np.arange(n_uf) * CF).astype(np.int32))
    if_off = jnp.asarray((np.arange(n_if) * CF).astype(np.int32))
    uf_idx = user_cat.astype(jnp.int32).T + uf_off[:, None]
    if_idx = item_cat.astype(jnp.int32).T + if_off[:, None]
    lincat_idx = jnp.concatenate(
        [uf_idx, if_idx + n_uf * CF], axis=0)
    ulin = user
    ilin = item + UV
    uirow = jnp.stack([ulin // 16, ilin // 16], axis=0)
    uilane = jnp.stack([ulin % 16, ilin % 16], axis=0)
    wlin2 = linear_W.reshape(total_lin // 16, 16)
    lincat_tab = linear_W[UV + IV:, 0]

    f32 = jnp.float32
    i32 = jnp.int32
    mesh = plsc.VectorSubcoreMesh(core_axis_name="c", subcore_axis_name="s",
                                  num_cores=NC, num_subcores=NS)
    sc_fn = pl.kernel(
        _sc_body,
        out_type=(jax.ShapeDtypeStruct((B, D), f32),
                  jax.ShapeDtypeStruct((B,), f32)),
        mesh=mesh,
        compiler_params=pltpu.CompilerParams(needs_layout_passes=False,
                                             use_tc_tiling_on_sc=False),
        scratch_types=[
            pltpu.VMEM(((n_uf + n_if) * CF,), f32),
            pltpu.VMEM((pw,), i32),
            pltpu.VMEM((pw,), i32),
            pltpu.VMEM((n_uf, pw), i32),
            pltpu.VMEM((n_if, pw), i32),
            pltpu.VMEM((n_uf + n_if, pw), i32),
            pltpu.VMEM((2, pw), i32),
            pltpu.VMEM((2, pw), i32),
            pltpu.VMEM(((2 + n_uf + n_if) * CH, D), f32),
            pltpu.VMEM((2 * CH, 16), f32),
            pltpu.VMEM((CH, D), f32),
            pltpu.VMEM((pw,), f32),
            pltpu.SemaphoreType.DMA,
        ],
    )
    cross, lin = sc_fn(user, item, uf_idx, if_idx, lincat_idx, uirow, uilane,
                       u_table, i_table, uf_table, if_table, wlin2,
                       lincat_tab)

    out = pl.pallas_call(
        _tc_body,
        out_shape=jax.ShapeDtypeStruct((B, 1), f32),
    )(cross, lin.reshape(B, 1), bn_g.reshape(1, D), bn_b.reshape(1, D),
      W1, b1.reshape(1, -1), g1.reshape(1, -1), bt1.reshape(1, -1),
      W2, b2.reshape(1, -1), g2.reshape(1, -1), bt2.reshape(1, -1),
      W3, b3.reshape(1, -1), linear_b.reshape(1, 1))
    return out[:, 0]
